# baseline (speedup 1.0000x reference)
import functools

import jax
import jax.numpy as jnp
from jax import lax
from jax.experimental import pallas as pl
from jax.experimental.pallas import tpu as pltpu
from jax.experimental.pallas import tpu_sc as plsc

N = 10000
D = 128
E = 320000
NSUB = 16
CHUNK = 128
CPS = 160
EPW = CPS * CHUNK
E_PAD = EPW * NSUB
IDXROWS = E_PAD // CHUNK
N_PAD = 10112
RPS = N_PAD // NSUB
NBUF = 2

_mesh = plsc.VectorSubcoreMesh(core_axis_name="c", subcore_axis_name="s")


IB = 16
NBLK = CPS // IB


def _sc_degrees(dst1, dst2, ones128, zD):
    @functools.partial(
        pl.kernel,
        out_type=[jax.ShapeDtypeStruct((N_PAD, D), jnp.float32)] * 2,
        mesh=_mesh,
        scratch_types=[
            pltpu.VMEM((IB, CHUNK), jnp.int32),
            pltpu.VMEM((CHUNK, D), jnp.float32),
            pltpu.VMEM_SHARED((N_PAD, D), jnp.float32),
            pltpu.SemaphoreType.DMA((4,)),
        ],
    )
    def deg_kernel(d1_hbm, d2_hbm, ones_hbm, z_hbm, o1_hbm, o2_hbm,
                   dst_v, ones_v, acc, ssem):
        cid = lax.axis_index("c")
        sid = lax.axis_index("s")
        pltpu.sync_copy(z_hbm, acc.at[pl.ds(sid * RPS, RPS)])
        pltpu.sync_copy(ones_hbm, ones_v)
        plsc.subcore_barrier()
        for core_val, d_hbm, o_hbm in ((0, d1_hbm, o1_hbm), (1, d2_hbm, o2_hbm)):
            @pl.when(cid == core_val)
            def _(d_hbm=d_hbm, o_hbm=o_hbm):
                @pl.loop(0, NBLK)
                def _(blk):
                    row0 = sid * CPS + blk * IB
                    pltpu.sync_copy(d_hbm.at[pl.ds(row0, IB)], dst_v)
                    descs = []
                    for j in range(IB):
                        descs.append(pltpu.async_copy(
                            ones_v, acc.at[dst_v.at[j]], ssem.at[j % 4],
                            add=True))
                    for dsc in descs:
                        dsc.wait()

                plsc.subcore_barrier()
                pltpu.sync_copy(acc.at[pl.ds(sid * RPS, RPS)],
                                o_hbm.at[pl.ds(sid * RPS, RPS)])

    return deg_kernel(dst1, dst2, ones128, zD)


def _sc_scatter(g1, src1, dst1, g2, src2, dst2, zD):
    @functools.partial(
        pl.kernel,
        out_type=[jax.ShapeDtypeStruct((N_PAD, D), jnp.float32)] * 2,
        mesh=_mesh,
        scratch_types=[
            pltpu.VMEM((IB, CHUNK), jnp.int32),
            pltpu.VMEM((IB, CHUNK), jnp.int32),
            pltpu.VMEM((NBUF, CHUNK, D), jnp.float32),
            pltpu.VMEM_SHARED((N_PAD, D), jnp.float32),
            pltpu.SemaphoreType.DMA((NBUF,)),
            pltpu.SemaphoreType.DMA((NBUF,)),
        ],
    )
    def scat_kernel(g1_hbm, s1_hbm, d1_hbm, g2_hbm, s2_hbm, d2_hbm,
                    z_hbm, o1_hbm, o2_hbm, src_v, dst_v, rows_v, acc,
                    gsem, ssem):
        cid = lax.axis_index("c")
        sid = lax.axis_index("s")
        pltpu.sync_copy(z_hbm, acc.at[pl.ds(sid * RPS, RPS)])
        plsc.subcore_barrier()
        for core_val, g_hbm, s_hbm, d_hbm, o_hbm in (
                (0, g1_hbm, s1_hbm, d1_hbm, o1_hbm),
                (1, g2_hbm, s2_hbm, d2_hbm, o2_hbm)):
            @pl.when(cid == core_val)
            def _(g_hbm=g_hbm, s_hbm=s_hbm, d_hbm=d_hbm, o_hbm=o_hbm):
                @pl.loop(0, NBLK)
                def _(blk):
                    row0 = sid * CPS + blk * IB
                    pltpu.sync_copy(s_hbm.at[pl.ds(row0, IB)], src_v)
                    pltpu.sync_copy(d_hbm.at[pl.ds(row0, IB)], dst_v)
                    for b in range(NBUF):
                        pltpu.async_copy(g_hbm.at[src_v.at[b]], rows_v.at[b],
                                         gsem.at[b])
                    for c in range(IB):
                        b = c % NBUF
                        pltpu.make_async_copy(g_hbm.at[src_v.at[0]],
                                              rows_v.at[b],
                                              gsem.at[b]).wait()
                        sdsc = pltpu.async_copy(rows_v.at[b],
                                                acc.at[dst_v.at[c]],
                                                ssem.at[b], add=True)
                        sdsc.wait()
                        if c + NBUF < IB:
                            pltpu.async_copy(g_hbm.at[src_v.at[c + NBUF]],
                                             rows_v.at[b], gsem.at[b])

                plsc.subcore_barrier()
                pltpu.sync_copy(acc.at[pl.ds(sid * RPS, RPS)],
                                o_hbm.at[pl.ds(sid * RPS, RPS)])

    return scat_kernel(g1, src1, dst1, g2, src2, dst2, zD)


_R = 2000


def _dot(a, b):
    return lax.dot_general(a, b, (((1,), (0,)), ((), ())),
                           precision=lax.Precision.HIGHEST,
                           preferred_element_type=jnp.float32)


def _tc_first(x, W, deg):
    def body(x_ref, w_ref, deg_ref, o_ref):
        dis = lax.rsqrt(deg_ref[:, 0:1] + 1.0)
        o_ref[...] = dis * _dot(x_ref[...], w_ref[...])

    return pl.pallas_call(
        body,
        grid=(N // _R,),
        in_specs=[pl.BlockSpec((_R, D), lambda i: (i, 0)),
                  pl.BlockSpec((D, D), lambda i: (0, 0)),
                  pl.BlockSpec((_R, D), lambda i: (i, 0))],
        out_specs=pl.BlockSpec((_R, D), lambda i: (i, 0)),
        out_shape=jax.ShapeDtypeStruct((N, D), jnp.float32),
    )(x, W, deg)


def _tc_mid(s, g, deg, b, W):
    def body(s_ref, g_ref, deg_ref, b_ref, w_ref, o_ref):
        dis = lax.rsqrt(deg_ref[:, 0:1] + 1.0)
        h = jnp.maximum(dis * (s_ref[...] + g_ref[...]) + b_ref[...], 0.0)
        o_ref[...] = dis * _dot(h, w_ref[...])

    return pl.pallas_call(
        body,
        grid=(N // _R,),
        in_specs=[pl.BlockSpec((_R, D), lambda i: (i, 0)),
                  pl.BlockSpec((_R, D), lambda i: (i, 0)),
                  pl.BlockSpec((_R, D), lambda i: (i, 0)),
                  pl.BlockSpec((1, D), lambda i: (0, 0)),
                  pl.BlockSpec((D, D), lambda i: (0, 0))],
        out_specs=pl.BlockSpec((_R, D), lambda i: (i, 0)),
        out_shape=jax.ShapeDtypeStruct((N, D), jnp.float32),
    )(s, g, deg, b.reshape(1, D), W)


def _tc_last(s, g, deg, b):
    def body(s_ref, g_ref, deg_ref, b_ref, o_ref):
        dis = lax.rsqrt(deg_ref[:, 0:1] + 1.0)
        o_ref[...] = dis * (s_ref[...] + g_ref[...]) + b_ref[...]

    return pl.pallas_call(
        body,
        grid=(N // _R,),
        in_specs=[pl.BlockSpec((_R, D), lambda i: (i, 0)),
                  pl.BlockSpec((_R, D), lambda i: (i, 0)),
                  pl.BlockSpec((_R, D), lambda i: (i, 0)),
                  pl.BlockSpec((1, D), lambda i: (0, 0))],
        out_specs=pl.BlockSpec((_R, D), lambda i: (i, 0)),
        out_shape=jax.ShapeDtypeStruct((N, D), jnp.float32),
    )(s, g, deg, b.reshape(1, D))


def kernel(x1, edge_index1, x2, edge_index2, args,
           W1_0, b1_0, W1_1, b1_1, W2_0, b2_0, W2_1, b2_1):
    del args
    pad_src = jnp.zeros((E_PAD - E,), jnp.int32)
    pad_dst = jnp.full((E_PAD - E,), N, jnp.int32)
    s1 = jnp.concatenate([edge_index1[0], pad_src]).reshape(IDXROWS, CHUNK)
    d1 = jnp.concatenate([edge_index1[1], pad_dst]).reshape(IDXROWS, CHUNK)
    s2 = jnp.concatenate([edge_index2[0], pad_src]).reshape(IDXROWS, CHUNK)
    d2 = jnp.concatenate([edge_index2[1], pad_dst]).reshape(IDXROWS, CHUNK)
    ones128 = jnp.ones((CHUNK, D), jnp.float32)
    zD = jnp.zeros((RPS, D), jnp.float32)

    dega1, dega2 = _sc_degrees(d1, d2, ones128, zD)
    deg1, deg2 = dega1[:N], dega2[:N]

    G11 = _tc_first(x1, W1_0, deg1)
    G21 = _tc_first(x2, W2_0, deg2)
    S11, S21 = _sc_scatter(G11, s1, d1, G21, s2, d2, zD)
    G12 = _tc_mid(S11[:N], G11, deg1, b1_0, W1_1)
    G22 = _tc_mid(S21[:N], G21, deg2, b2_0, W2_1)
    S12, S22 = _sc_scatter(G12, s1, d1, G22, s2, d2, zD)
    out1 = _tc_last(S12[:N], G12, deg1, b1_1)
    out2 = _tc_last(S22[:N], G22, deg2, b2_1)
    return (out1, out2)


# speedup vs baseline: 1.0487x; 1.0487x over previous
import dataclasses
import functools

import jax
import jax.numpy as jnp
from jax import lax
from jax.experimental import pallas as pl
from jax.experimental.pallas import tpu as pltpu
from jax.experimental.pallas import tpu_sc as plsc

N = 10000
D = 128
E = 320000
NSUB = 16
CHUNK = 128
CPS = 160
EPW = CPS * CHUNK
E_PAD = EPW * NSUB
IDXROWS = E_PAD // CHUNK
N_PAD = 10112
RPS = N_PAD // NSUB
NBUF = 2

_mesh = plsc.VectorSubcoreMesh(core_axis_name="c", subcore_axis_name="s")

_sc_params = pltpu.CompilerParams()
if "needs_layout_passes" in pltpu.CompilerParams.__dataclass_fields__:
    _sc_params = dataclasses.replace(_sc_params, needs_layout_passes=False)


IB = 16
NBLK = CPS // IB


def _sc_degrees(dst1, dst2):
    @functools.partial(
        pl.kernel,
        out_type=[jax.ShapeDtypeStruct((NSUB, 1, N_PAD), jnp.float32)] * 2,
        mesh=_mesh,
        compiler_params=_sc_params,
        scratch_types=[
            pltpu.VMEM((IB, CHUNK), jnp.int32),
            pltpu.VMEM((1, N_PAD), jnp.float32),
        ],
    )
    def deg_kernel(d1_hbm, d2_hbm, o1_hbm, o2_hbm, idx_v, hist_v):
        cid = lax.axis_index("c")
        sid = lax.axis_index("s")

        @pl.loop(0, N_PAD // 16)
        def _(i):
            hist_v[0, pl.ds(i * 16, 16)] = jnp.zeros((16,), jnp.float32)

        zeros16 = jnp.zeros((16,), jnp.int32)
        ones16f = jnp.ones((16,), jnp.float32)
        for core_val, d_hbm, o_hbm in ((0, d1_hbm, o1_hbm), (1, d2_hbm, o2_hbm)):
            @pl.when(cid == core_val)
            def _(d_hbm=d_hbm, o_hbm=o_hbm):
                @pl.loop(0, NBLK)
                def _(blk):
                    row0 = sid * CPS + blk * IB
                    pltpu.sync_copy(d_hbm.at[pl.ds(row0, IB)], idx_v)

                    @pl.loop(0, IB)
                    def _(j):
                        for q in range(CHUNK // 16):
                            vals = idx_v[j, pl.ds(q * 16, 16)]
                            plsc.addupdate_scatter(hist_v, [zeros16, vals],
                                                   ones16f)

                pltpu.sync_copy(hist_v, o_hbm.at[sid])

    return deg_kernel(dst1, dst2)


def _sc_scatter(g1, src1, dst1, g2, src2, dst2, zD):
    @functools.partial(
        pl.kernel,
        out_type=[jax.ShapeDtypeStruct((N_PAD, D), jnp.float32)] * 2,
        mesh=_mesh,
        scratch_types=[
            pltpu.VMEM((IB, CHUNK), jnp.int32),
            pltpu.VMEM((IB, CHUNK), jnp.int32),
            pltpu.VMEM((NBUF, CHUNK, D), jnp.float32),
            pltpu.VMEM_SHARED((N_PAD, D), jnp.float32),
            pltpu.SemaphoreType.DMA((NBUF,)),
            pltpu.SemaphoreType.DMA((NBUF,)),
        ],
    )
    def scat_kernel(g1_hbm, s1_hbm, d1_hbm, g2_hbm, s2_hbm, d2_hbm,
                    z_hbm, o1_hbm, o2_hbm, src_v, dst_v, rows_v, acc,
                    gsem, ssem):
        cid = lax.axis_index("c")
        sid = lax.axis_index("s")
        pltpu.sync_copy(z_hbm, acc.at[pl.ds(sid * RPS, RPS)])
        plsc.subcore_barrier()
        for core_val, g_hbm, s_hbm, d_hbm, o_hbm in (
                (0, g1_hbm, s1_hbm, d1_hbm, o1_hbm),
                (1, g2_hbm, s2_hbm, d2_hbm, o2_hbm)):
            @pl.when(cid == core_val)
            def _(g_hbm=g_hbm, s_hbm=s_hbm, d_hbm=d_hbm, o_hbm=o_hbm):
                @pl.loop(0, NBLK)
                def _(blk):
                    row0 = sid * CPS + blk * IB
                    pltpu.sync_copy(s_hbm.at[pl.ds(row0, IB)], src_v)
                    pltpu.sync_copy(d_hbm.at[pl.ds(row0, IB)], dst_v)
                    for b in range(NBUF):
                        pltpu.async_copy(g_hbm.at[src_v.at[b]], rows_v.at[b],
                                         gsem.at[b])
                    for c in range(IB):
                        b = c % NBUF
                        pltpu.make_async_copy(g_hbm.at[src_v.at[0]],
                                              rows_v.at[b],
                                              gsem.at[b]).wait()
                        sdsc = pltpu.async_copy(rows_v.at[b],
                                                acc.at[dst_v.at[c]],
                                                ssem.at[b], add=True)
                        sdsc.wait()
                        if c + NBUF < IB:
                            pltpu.async_copy(g_hbm.at[src_v.at[c + NBUF]],
                                             rows_v.at[b], gsem.at[b])

                plsc.subcore_barrier()
                pltpu.sync_copy(acc.at[pl.ds(sid * RPS, RPS)],
                                o_hbm.at[pl.ds(sid * RPS, RPS)])

    return scat_kernel(g1, src1, dst1, g2, src2, dst2, zD)


_R = 2000


def _dot(a, b):
    return lax.dot_general(a, b, (((1,), (0,)), ((), ())),
                           precision=lax.Precision.HIGHEST,
                           preferred_element_type=jnp.float32)


def _tc_dis(deg16):
    def body(deg_ref, o_ref):
        dsum = jnp.sum(deg_ref[...], axis=0)
        o_ref[...] = lax.rsqrt(dsum + 1.0)[:, None]

    return pl.pallas_call(
        body,
        grid=(N_PAD // 128,),
        in_specs=[pl.BlockSpec((NSUB, 128), lambda i: (0, i))],
        out_specs=pl.BlockSpec((128, 1), lambda i: (i, 0)),
        out_shape=jax.ShapeDtypeStruct((N_PAD, 1), jnp.float32),
    )(deg16)


def _tc_first(x, W, dis):
    def body(x_ref, w_ref, dis_ref, o_ref):
        o_ref[...] = dis_ref[...] * _dot(x_ref[...], w_ref[...])

    return pl.pallas_call(
        body,
        grid=(N // _R,),
        in_specs=[pl.BlockSpec((_R, D), lambda i: (i, 0)),
                  pl.BlockSpec((D, D), lambda i: (0, 0)),
                  pl.BlockSpec((_R, 1), lambda i: (i, 0))],
        out_specs=pl.BlockSpec((_R, D), lambda i: (i, 0)),
        out_shape=jax.ShapeDtypeStruct((N, D), jnp.float32),
    )(x, W, dis)


def _tc_mid(s, g, dis, b, W):
    def body(s_ref, g_ref, dis_ref, b_ref, w_ref, o_ref):
        dis = dis_ref[...]
        h = jnp.maximum(dis * (s_ref[...] + g_ref[...]) + b_ref[...], 0.0)
        o_ref[...] = dis * _dot(h, w_ref[...])

    return pl.pallas_call(
        body,
        grid=(N // _R,),
        in_specs=[pl.BlockSpec((_R, D), lambda i: (i, 0)),
                  pl.BlockSpec((_R, D), lambda i: (i, 0)),
                  pl.BlockSpec((_R, 1), lambda i: (i, 0)),
                  pl.BlockSpec((1, D), lambda i: (0, 0)),
                  pl.BlockSpec((D, D), lambda i: (0, 0))],
        out_specs=pl.BlockSpec((_R, D), lambda i: (i, 0)),
        out_shape=jax.ShapeDtypeStruct((N, D), jnp.float32),
    )(s, g, dis, b.reshape(1, D), W)


def _tc_last(s, g, dis, b):
    def body(s_ref, g_ref, dis_ref, b_ref, o_ref):
        o_ref[...] = dis_ref[...] * (s_ref[...] + g_ref[...]) + b_ref[...]

    return pl.pallas_call(
        body,
        grid=(N // _R,),
        in_specs=[pl.BlockSpec((_R, D), lambda i: (i, 0)),
                  pl.BlockSpec((_R, D), lambda i: (i, 0)),
                  pl.BlockSpec((_R, 1), lambda i: (i, 0)),
                  pl.BlockSpec((1, D), lambda i: (0, 0))],
        out_specs=pl.BlockSpec((_R, D), lambda i: (i, 0)),
        out_shape=jax.ShapeDtypeStruct((N, D), jnp.float32),
    )(s, g, dis, b.reshape(1, D))


def kernel(x1, edge_index1, x2, edge_index2, args,
           W1_0, b1_0, W1_1, b1_1, W2_0, b2_0, W2_1, b2_1):
    del args
    pad_src = jnp.zeros((E_PAD - E,), jnp.int32)
    pad_dst = jnp.full((E_PAD - E,), N, jnp.int32)
    s1 = jnp.concatenate([edge_index1[0], pad_src]).reshape(IDXROWS, CHUNK)
    d1 = jnp.concatenate([edge_index1[1], pad_dst]).reshape(IDXROWS, CHUNK)
    s2 = jnp.concatenate([edge_index2[0], pad_src]).reshape(IDXROWS, CHUNK)
    d2 = jnp.concatenate([edge_index2[1], pad_dst]).reshape(IDXROWS, CHUNK)
    zD = jnp.zeros((RPS, D), jnp.float32)

    dega1, dega2 = _sc_degrees(d1, d2)
    dis1 = _tc_dis(dega1.reshape(NSUB, N_PAD))[:N]
    dis2 = _tc_dis(dega2.reshape(NSUB, N_PAD))[:N]

    G11 = _tc_first(x1, W1_0, dis1)
    G21 = _tc_first(x2, W2_0, dis2)
    S11, S21 = _sc_scatter(G11, s1, d1, G21, s2, d2, zD)
    G12 = _tc_mid(S11[:N], G11, dis1, b1_0, W1_1)
    G22 = _tc_mid(S21[:N], G21, dis2, b2_0, W2_1)
    S12, S22 = _sc_scatter(G12, s1, d1, G22, s2, d2, zD)
    out1 = _tc_last(S12[:N], G12, dis1, b1_1)
    out2 = _tc_last(S22[:N], G22, dis2, b2_1)
    return (out1, out2)
